# Initial kernel scaffold; baseline (speedup 1.0000x reference)
import jax
import jax.numpy as jnp
from jax.experimental import pallas as pl


def kernel(node_feats, positions, edge_index, edge_attributes, params):
    raise NotImplementedError("write your pallas kernel here")


# speedup vs baseline: 2.9730x; 2.9730x over previous
import functools

import jax
import jax.numpy as jnp
from jax import lax
from jax.experimental import pallas as pl
from jax.experimental.pallas import tpu as pltpu
from jax.experimental.pallas import tpu_sc as plsc

H = 128
XW = 16
NC = 2
NS = 16
NW = NC * NS
CHUNK = 80
BN = 1000
BE = 2000
F32 = jnp.float32


def _sigmoid(v):
    return 1.0 / (1.0 + jnp.exp(-v))


def _silu(v):
    return v * _sigmoid(v)


def _dot(a, b):
    return jnp.dot(a, b, preferred_element_type=F32)


def _sc_mesh():
    return plsc.VectorSubcoreMesh(core_axis_name="c", subcore_axis_name="s")


def _sc_gather(hd, hs, xpad, src, dst):
    n, _ = hd.shape
    e = src.shape[0]
    ew = e // NW
    nch = ew // CHUNK

    @functools.partial(
        pl.kernel,
        out_type=(
            jax.ShapeDtypeStruct((e, H), F32),
            jax.ShapeDtypeStruct((e, H), F32),
            jax.ShapeDtypeStruct((e, XW), F32),
            jax.ShapeDtypeStruct((e, XW), F32),
        ),
        mesh=_sc_mesh(),
        scratch_types=[
            pltpu.VMEM((CHUNK,), jnp.int32),
            pltpu.VMEM((CHUNK,), jnp.int32),
            pltpu.VMEM((CHUNK, H), F32),
            pltpu.VMEM((CHUNK, H), F32),
            pltpu.VMEM((CHUNK, XW), F32),
            pltpu.VMEM((CHUNK, XW), F32),
            pltpu.SemaphoreType.DMA,
            pltpu.SemaphoreType.DMA,
            pltpu.SemaphoreType.DMA,
            pltpu.SemaphoreType.DMA,
        ],
        compiler_params=pltpu.CompilerParams(use_tc_tiling_on_sc=False),
    )
    def k(hd_h, hs_h, xp_h, src_h, dst_h, gd_h, gs_h, xs_h, xd_h,
          src_v, dst_v, rd_v, rs_v, xs_v, xd_v, s1, s2, s3, s4):
        wid = lax.axis_index("s") * NC + lax.axis_index("c")

        def chunk(i, carry):
            base = wid * ew + i * CHUNK
            pltpu.sync_copy(src_h.at[pl.ds(base, CHUNK)], src_v)
            pltpu.sync_copy(dst_h.at[pl.ds(base, CHUNK)], dst_v)
            c1 = pltpu.async_copy(hd_h.at[dst_v], rd_v, s1)
            c2 = pltpu.async_copy(hs_h.at[src_v], rs_v, s2)
            c3 = pltpu.async_copy(xp_h.at[src_v], xs_v, s3)
            c4 = pltpu.async_copy(xp_h.at[dst_v], xd_v, s4)
            c1.wait()
            c2.wait()
            c3.wait()
            c4.wait()
            pltpu.sync_copy(rd_v, gd_h.at[pl.ds(base, CHUNK)])
            pltpu.sync_copy(rs_v, gs_h.at[pl.ds(base, CHUNK)])
            pltpu.sync_copy(xs_v, xs_h.at[pl.ds(base, CHUNK)])
            pltpu.sync_copy(xd_v, xd_h.at[pl.ds(base, CHUNK)])
            return carry

        lax.fori_loop(0, nch, chunk, 0)

    return k(hd, hs, xpad, src, dst)


def _sc_scatter(m, trans, dst, z128, z16):
    e = dst.shape[0]
    n = z128.shape[0]
    ew = e // NW
    nch = ew // CHUNK
    rpt = n // NS

    @functools.partial(
        pl.kernel,
        out_type=(
            jax.ShapeDtypeStruct((NC, n, H), F32),
            jax.ShapeDtypeStruct((NC, n, XW), F32),
        ),
        mesh=_sc_mesh(),
        scratch_types=[
            pltpu.VMEM((CHUNK, H), F32),
            pltpu.VMEM((CHUNK, XW), F32),
            pltpu.VMEM((CHUNK,), jnp.int32),
            pltpu.VMEM_SHARED((n, H), F32),
            pltpu.VMEM_SHARED((n, XW), F32),
        ],
        compiler_params=pltpu.CompilerParams(use_tc_tiling_on_sc=False),
    )
    def k(m_h, t_h, dst_h, z128_h, z16_h, mago_h, aggo_h,
          m_v, t_v, dst_v, mag_sh, agg_sh):
        cid = lax.axis_index("c")
        sid = lax.axis_index("s")
        wid = sid * NC + cid
        row0 = sid * rpt
        pltpu.sync_copy(z128_h.at[pl.ds(row0, rpt)], mag_sh.at[pl.ds(row0, rpt)])
        pltpu.sync_copy(z16_h.at[pl.ds(row0, rpt)], agg_sh.at[pl.ds(row0, rpt)])
        plsc.subcore_barrier()

        def chunk(i, carry):
            base = wid * ew + i * CHUNK
            pltpu.sync_copy(dst_h.at[pl.ds(base, CHUNK)], dst_v)
            pltpu.sync_copy(m_h.at[pl.ds(base, CHUNK)], m_v)
            pltpu.sync_copy(t_h.at[pl.ds(base, CHUNK)], t_v)
            pltpu.sync_copy(m_v, mag_sh.at[dst_v], add=True)
            pltpu.sync_copy(t_v, agg_sh.at[dst_v], add=True)
            return carry

        lax.fori_loop(0, nch, chunk, 0)
        plsc.subcore_barrier()
        pltpu.sync_copy(mag_sh.at[pl.ds(row0, rpt)],
                        mago_h.at[cid, pl.ds(row0, rpt)])
        pltpu.sync_copy(agg_sh.at[pl.ds(row0, rpt)],
                        aggo_h.at[cid, pl.ds(row0, rpt)])

    return k(m, trans, dst, z128, z16)


def _tc0(feats2d, x, emb, wemb, bemb, w1d, w1s, interpret=False):
    n = feats2d.shape[0]
    nv = emb.shape[0]

    def body(f_ref, x_ref, emb_ref, wemb_ref, bemb_ref, w1d_ref, w1s_ref,
             h_ref, hd_ref, hs_ref, xp_ref):
        f = f_ref[...]
        iota = lax.broadcasted_iota(jnp.int32, (1, nv), 1)
        onehot = (f == iota).astype(F32)
        embw = _dot(emb_ref[...], wemb_ref[...])
        h = _dot(onehot, embw) + bemb_ref[...]
        h_ref[...] = h
        hd_ref[...] = _dot(h, w1d_ref[...])
        hs_ref[...] = _dot(h, w1s_ref[...])
        xx = x_ref[...]
        xp_ref[...] = jnp.concatenate(
            [xx, jnp.zeros((xx.shape[0], XW - 3), F32)], axis=1)

    grid = (n // BN,)
    full = lambda s: pl.BlockSpec(s, lambda i: (0,) * len(s))
    return pl.pallas_call(
        body,
        grid=grid,
        in_specs=[
            pl.BlockSpec((BN, 1), lambda i: (i, 0)),
            pl.BlockSpec((BN, 3), lambda i: (i, 0)),
            full((nv, H)),
            full((H, H)),
            full((1, H)),
            full((H, H)),
            full((H, H)),
        ],
        out_specs=[
            pl.BlockSpec((BN, H), lambda i: (i, 0)),
            pl.BlockSpec((BN, H), lambda i: (i, 0)),
            pl.BlockSpec((BN, H), lambda i: (i, 0)),
            pl.BlockSpec((BN, XW), lambda i: (i, 0)),
        ],
        out_shape=[
            jax.ShapeDtypeStruct((n, H), F32),
            jax.ShapeDtypeStruct((n, H), F32),
            jax.ShapeDtypeStruct((n, H), F32),
            jax.ShapeDtypeStruct((n, XW), F32),
        ],
        interpret=interpret,
    )(feats2d, x, emb, wemb, bemb, w1d, w1s)


def _tc_edge(gd, gs, xs, xd, ea, wx, wp1, bp1, wp2, interpret=False):
    e = gd.shape[0]
    de = ea.shape[1]

    def body(gd_ref, gs_ref, xs_ref, xd_ref, ea_ref, wx_ref, wp1_ref,
             bp1_ref, wp2_ref, m_ref, t_ref):
        d = xs_ref[...] - xd_ref[...]
        d2 = jnp.sum(d * d, axis=1, keepdims=True)
        ones = jnp.ones((d.shape[0], 1), F32)
        zer = jnp.zeros((d.shape[0], 8 - 2 - de), F32)
        feat = jnp.concatenate([d2, ea_ref[...], ones, zer], axis=1)
        mpre = gd_ref[...] + gs_ref[...] + _dot(feat, wx_ref[...])
        m = _silu(mpre)
        m_ref[...] = m
        q = _dot(m, wp1_ref[...]) + bp1_ref[...]
        q = _silu(q)
        p = jnp.sum(q * wp2_ref[...], axis=1, keepdims=True)
        invd = 1.0 / (jnp.sqrt(d2) + 1.0)
        t = d * (p * invd)
        col3 = (lax.broadcasted_iota(jnp.int32, (1, XW), 1) == 3).astype(F32)
        t_ref[...] = t + col3

    grid = (e // BE,)
    full = lambda s: pl.BlockSpec(s, lambda i: (0,) * len(s))
    return pl.pallas_call(
        body,
        grid=grid,
        in_specs=[
            pl.BlockSpec((BE, H), lambda i: (i, 0)),
            pl.BlockSpec((BE, H), lambda i: (i, 0)),
            pl.BlockSpec((BE, XW), lambda i: (i, 0)),
            pl.BlockSpec((BE, XW), lambda i: (i, 0)),
            pl.BlockSpec((BE, de), lambda i: (i, 0)),
            full((8, H)),
            full((H, 64)),
            full((1, 64)),
            full((1, 64)),
        ],
        out_specs=[
            pl.BlockSpec((BE, H), lambda i: (i, 0)),
            pl.BlockSpec((BE, XW), lambda i: (i, 0)),
        ],
        out_shape=[
            jax.ShapeDtypeStruct((e, H), F32),
            jax.ShapeDtypeStruct((e, XW), F32),
        ],
        interpret=interpret,
    )(gd, gs, xs, xd, ea, wx, wp1, bp1, wp2)


def _tc_node(h, xpad, magg2, agg2, wf1a, wf1b, bf1, wf2, bf2,
             w1d=None, w1s=None, last=False, interpret=False):
    n = h.shape[0]

    def body(*refs):
        if last:
            (h_ref, xp_ref, mg_ref, ag_ref, wf1a_ref, wf1b_ref, bf1_ref,
             wf2_ref, bf2_ref, ho_ref, xo_ref) = refs
        else:
            (h_ref, xp_ref, mg_ref, ag_ref, wf1a_ref, wf1b_ref, bf1_ref,
             wf2_ref, bf2_ref, w1d_ref, w1s_ref,
             ho_ref, xo_ref, hd_ref, hs_ref) = refs
        magg = mg_ref[0] + mg_ref[1]
        a = ag_ref[0] + ag_ref[1]
        cnt = jnp.maximum(a[:, 3:4], 1.0)
        posmask = (lax.broadcasted_iota(jnp.int32, (1, XW), 1) < 3).astype(F32)
        xp = xp_ref[...] + (a * posmask) / cnt
        h_ = h_ref[...]
        f = _dot(h_, wf1a_ref[...]) + _dot(magg, wf1b_ref[...]) + bf1_ref[...]
        f = _silu(f)
        f = _dot(f, wf2_ref[...]) + bf2_ref[...]
        hn = h_ + f
        if last:
            ho_ref[...] = hn
            xo_ref[...] = xp[:, :3]
        else:
            hn = _silu(hn)
            ho_ref[...] = hn
            xo_ref[...] = xp
            hd_ref[...] = _dot(hn, w1d_ref[...])
            hs_ref[...] = _dot(hn, w1s_ref[...])

    grid = (n // BN,)
    full = lambda s: pl.BlockSpec(s, lambda i: (0,) * len(s))
    in_specs = [
        pl.BlockSpec((BN, H), lambda i: (i, 0)),
        pl.BlockSpec((BN, XW), lambda i: (i, 0)),
        pl.BlockSpec((NC, BN, H), lambda i: (0, i, 0)),
        pl.BlockSpec((NC, BN, XW), lambda i: (0, i, 0)),
        full((H, H)),
        full((H, H)),
        full((1, H)),
        full((H, H)),
        full((1, H)),
    ]
    args = [h, xpad, magg2, agg2, wf1a, wf1b, bf1, wf2, bf2]
    if last:
        out_specs = [
            pl.BlockSpec((BN, H), lambda i: (i, 0)),
            pl.BlockSpec((BN, 3), lambda i: (i, 0)),
        ]
        out_shape = [
            jax.ShapeDtypeStruct((n, H), F32),
            jax.ShapeDtypeStruct((n, 3), F32),
        ]
    else:
        in_specs += [full((H, H)), full((H, H))]
        args += [w1d, w1s]
        out_specs = [
            pl.BlockSpec((BN, H), lambda i: (i, 0)),
            pl.BlockSpec((BN, XW), lambda i: (i, 0)),
            pl.BlockSpec((BN, H), lambda i: (i, 0)),
            pl.BlockSpec((BN, H), lambda i: (i, 0)),
        ]
        out_shape = [
            jax.ShapeDtypeStruct((n, H), F32),
            jax.ShapeDtypeStruct((n, XW), F32),
            jax.ShapeDtypeStruct((n, H), F32),
            jax.ShapeDtypeStruct((n, H), F32),
        ]
    return pl.pallas_call(
        body,
        grid=grid,
        in_specs=in_specs,
        out_specs=out_specs,
        out_shape=out_shape,
        interpret=interpret,
    )(*args)


def _layer_weights(layer):
    w1 = layer["edge"][0]["w"]
    b1 = layer["edge"][0]["b"]
    de = w1.shape[0] - 2 * H - 1
    w1d = w1[0:H]
    w1s = w1[H:2 * H]
    wx = jnp.concatenate(
        [w1[2 * H:2 * H + 1], w1[2 * H + 1:], b1[None],
         jnp.zeros((8 - 2 - de, w1.shape[1]), F32)], axis=0)
    wp1 = layer["pos"][0]["w"]
    bp1 = layer["pos"][0]["b"][None]
    wp2 = layer["pos"][1]["w"].T
    wf1 = layer["feat"][0]["w"]
    wf1a = wf1[:H]
    wf1b = wf1[H:]
    bf1 = layer["feat"][0]["b"][None]
    wf2 = layer["feat"][1]["w"]
    bf2 = layer["feat"][1]["b"][None]
    return w1d, w1s, wx, wp1, bp1, wp2, wf1a, wf1b, bf1, wf2, bf2


def kernel(node_feats, positions, edge_index, edge_attributes, params):
    n = node_feats.shape[0]
    e = edge_index.shape[1]
    src = edge_index[0]
    dst = edge_index[1]
    layers = params["layers"]
    depth = len(layers)
    lw = [_layer_weights(L) for L in layers]

    h, hd, hs, xpad = _tc0(
        node_feats.reshape(n, 1), positions, params["emb"],
        params["in_embed"]["w"], params["in_embed"]["b"][None],
        lw[0][0], lw[0][1])

    z128 = jnp.zeros((n, H), F32)
    z16 = jnp.zeros((n, XW), F32)

    for i in range(depth):
        w1d, w1s, wx, wp1, bp1, wp2, wf1a, wf1b, bf1, wf2, bf2 = lw[i]
        gd, gs, xs, xd = _sc_gather(hd, hs, xpad, src, dst)
        m, trans = _tc_edge(gd, gs, xs, xd, edge_attributes, wx, wp1, bp1, wp2)
        magg2, agg2 = _sc_scatter(m, trans, dst, z128, z16)
        if i == depth - 1:
            h, x = _tc_node(h, xpad, magg2, agg2, wf1a, wf1b, bf1, wf2, bf2,
                            last=True)
        else:
            h, xpad, hd, hs = _tc_node(
                h, xpad, magg2, agg2, wf1a, wf1b, bf1, wf2, bf2,
                w1d=lw[i + 1][0], w1s=lw[i + 1][1], last=False)
    return h, x


# speedup vs baseline: 3.3866x; 1.1391x over previous
import functools

import jax
import jax.numpy as jnp
from jax import lax
from jax.experimental import pallas as pl
from jax.experimental.pallas import tpu as pltpu
from jax.experimental.pallas import tpu_sc as plsc

H = 128
XW = 16
NC = 2
NS = 16
NW = NC * NS
CHUNK = 80
BN = 1000
BE = 2000
F32 = jnp.float32


def _sigmoid(v):
    return 1.0 / (1.0 + jnp.exp(-v))


def _silu(v):
    return v * _sigmoid(v)


def _dot(a, b):
    return jnp.dot(a, b, preferred_element_type=F32)


def _sc_mesh():
    return plsc.VectorSubcoreMesh(core_axis_name="c", subcore_axis_name="s")


def _sc_gather(hd, hs, xpad, src, dst):
    n, _ = hd.shape
    e = src.shape[0]
    ew = e // NW
    nch = ew // CHUNK
    assert nch % 2 == 1 and nch >= 3

    buf = lambda: [
        pltpu.VMEM((CHUNK,), jnp.int32),
        pltpu.VMEM((CHUNK,), jnp.int32),
        pltpu.VMEM((CHUNK, H), F32),
        pltpu.VMEM((CHUNK, H), F32),
        pltpu.VMEM((CHUNK, XW), F32),
        pltpu.VMEM((CHUNK, XW), F32),
        pltpu.SemaphoreType.DMA,
    ]

    @functools.partial(
        pl.kernel,
        out_type=(
            jax.ShapeDtypeStruct((e, H), F32),
            jax.ShapeDtypeStruct((e, XW), F32),
            jax.ShapeDtypeStruct((e, XW), F32),
        ),
        mesh=_sc_mesh(),
        scratch_types=buf() + buf(),
        compiler_params=pltpu.CompilerParams(use_tc_tiling_on_sc=False),
    )
    def k(hd_h, hs_h, xp_h, src_h, dst_h, gsum_h, xs_h, xd_h, *scr):
        seta, setb = scr[:7], scr[7:]
        wid = lax.axis_index("s") * NC + lax.axis_index("c")

        def issue(c, st):
            sv, dv, rd, rs, xs, xd, sem = st
            base = wid * ew + c * CHUNK
            pltpu.sync_copy(src_h.at[pl.ds(base, CHUNK)], sv)
            pltpu.sync_copy(dst_h.at[pl.ds(base, CHUNK)], dv)
            pltpu.async_copy(hd_h.at[dv], rd, sem)
            pltpu.async_copy(hs_h.at[sv], rs, sem)
            pltpu.async_copy(xp_h.at[sv], xs, sem)
            pltpu.async_copy(xp_h.at[dv], xd, sem)

        def process(c, st):
            sv, dv, rd, rs, xs, xd, sem = st
            base = wid * ew + c * CHUNK
            pltpu.make_async_copy(hd_h.at[dv], rd, sem).wait()
            pltpu.make_async_copy(hs_h.at[sv], rs, sem).wait()
            pltpu.make_async_copy(xp_h.at[sv], xs, sem).wait()
            pltpu.make_async_copy(xp_h.at[dv], xd, sem).wait()

            def row(r, carry):
                for j in range(H // 16):
                    sl = pl.ds(j * 16, 16)
                    rd[r, sl] = rd[r, sl] + rs[r, sl]
                return carry

            lax.fori_loop(0, CHUNK, row, 0)
            pltpu.sync_copy(rd, gsum_h.at[pl.ds(base, CHUNK)])
            pltpu.sync_copy(xs, xs_h.at[pl.ds(base, CHUNK)])
            pltpu.sync_copy(xd, xd_h.at[pl.ds(base, CHUNK)])

        issue(0, seta)

        def body(i, carry):
            issue(2 * i + 1, setb)
            process(2 * i, seta)
            issue(2 * i + 2, seta)
            process(2 * i + 1, setb)
            return carry

        lax.fori_loop(0, (nch - 1) // 2, body, 0)
        process(nch - 1, seta)

    return k(hd, hs, xpad, src, dst)


def _sc_scatter(m, trans, dst, z128, z16):
    e = dst.shape[0]
    n = z128.shape[0]
    ew = e // NW
    nch = ew // CHUNK
    rpt = n // NS

    @functools.partial(
        pl.kernel,
        out_type=(
            jax.ShapeDtypeStruct((NC, n, H), F32),
            jax.ShapeDtypeStruct((NC, n, XW), F32),
        ),
        mesh=_sc_mesh(),
        scratch_types=[
            pltpu.VMEM((CHUNK, H), F32),
            pltpu.VMEM((CHUNK, XW), F32),
            pltpu.VMEM((CHUNK,), jnp.int32),
            pltpu.VMEM_SHARED((n, H), F32),
            pltpu.VMEM_SHARED((n, XW), F32),
        ],
        compiler_params=pltpu.CompilerParams(use_tc_tiling_on_sc=False),
    )
    def k(m_h, t_h, dst_h, z128_h, z16_h, mago_h, aggo_h,
          m_v, t_v, dst_v, mag_sh, agg_sh):
        cid = lax.axis_index("c")
        sid = lax.axis_index("s")
        wid = sid * NC + cid
        row0 = sid * rpt
        pltpu.sync_copy(z128_h.at[pl.ds(row0, rpt)], mag_sh.at[pl.ds(row0, rpt)])
        pltpu.sync_copy(z16_h.at[pl.ds(row0, rpt)], agg_sh.at[pl.ds(row0, rpt)])
        plsc.subcore_barrier()

        def chunk(i, carry):
            base = wid * ew + i * CHUNK
            pltpu.sync_copy(dst_h.at[pl.ds(base, CHUNK)], dst_v)
            pltpu.sync_copy(m_h.at[pl.ds(base, CHUNK)], m_v)
            pltpu.sync_copy(t_h.at[pl.ds(base, CHUNK)], t_v)
            pltpu.sync_copy(m_v, mag_sh.at[dst_v], add=True)
            pltpu.sync_copy(t_v, agg_sh.at[dst_v], add=True)
            return carry

        lax.fori_loop(0, nch, chunk, 0)
        plsc.subcore_barrier()
        pltpu.sync_copy(mag_sh.at[pl.ds(row0, rpt)],
                        mago_h.at[cid, pl.ds(row0, rpt)])
        pltpu.sync_copy(agg_sh.at[pl.ds(row0, rpt)],
                        aggo_h.at[cid, pl.ds(row0, rpt)])

    return k(m, trans, dst, z128, z16)


def _tc0(feats2d, x, emb, wemb, bemb, w1d, w1s, interpret=False):
    n = feats2d.shape[0]
    nv = emb.shape[0]

    def body(f_ref, x_ref, emb_ref, wemb_ref, bemb_ref, w1d_ref, w1s_ref,
             h_ref, hd_ref, hs_ref, xp_ref):
        f = f_ref[...]
        iota = lax.broadcasted_iota(jnp.int32, (1, nv), 1)
        onehot = (f == iota).astype(F32)
        embw = _dot(emb_ref[...], wemb_ref[...])
        h = _dot(onehot, embw) + bemb_ref[...]
        h_ref[...] = h
        hd_ref[...] = _dot(h, w1d_ref[...])
        hs_ref[...] = _dot(h, w1s_ref[...])
        xx = x_ref[...]
        xp_ref[...] = jnp.concatenate(
            [xx, jnp.zeros((xx.shape[0], XW - 3), F32)], axis=1)

    grid = (n // BN,)
    full = lambda s: pl.BlockSpec(s, lambda i: (0,) * len(s))
    return pl.pallas_call(
        body,
        grid=grid,
        in_specs=[
            pl.BlockSpec((BN, 1), lambda i: (i, 0)),
            pl.BlockSpec((BN, 3), lambda i: (i, 0)),
            full((nv, H)),
            full((H, H)),
            full((1, H)),
            full((H, H)),
            full((H, H)),
        ],
        out_specs=[
            pl.BlockSpec((BN, H), lambda i: (i, 0)),
            pl.BlockSpec((BN, H), lambda i: (i, 0)),
            pl.BlockSpec((BN, H), lambda i: (i, 0)),
            pl.BlockSpec((BN, XW), lambda i: (i, 0)),
        ],
        out_shape=[
            jax.ShapeDtypeStruct((n, H), F32),
            jax.ShapeDtypeStruct((n, H), F32),
            jax.ShapeDtypeStruct((n, H), F32),
            jax.ShapeDtypeStruct((n, XW), F32),
        ],
        interpret=interpret,
    )(feats2d, x, emb, wemb, bemb, w1d, w1s)


def _tc_edge(gsum, xs, xd, ea, wx, wp1, bp1, wp2, interpret=False):
    e = gsum.shape[0]
    de = ea.shape[1]

    def body(gsum_ref, xs_ref, xd_ref, ea_ref, wx_ref, wp1_ref,
             bp1_ref, wp2_ref, m_ref, t_ref):
        d = xs_ref[...] - xd_ref[...]
        d2 = jnp.sum(d * d, axis=1, keepdims=True)
        ones = jnp.ones((d.shape[0], 1), F32)
        zer = jnp.zeros((d.shape[0], 8 - 2 - de), F32)
        feat = jnp.concatenate([d2, ea_ref[...], ones, zer], axis=1)
        mpre = gsum_ref[...] + _dot(feat, wx_ref[...])
        m = _silu(mpre)
        m_ref[...] = m
        q = _dot(m, wp1_ref[...]) + bp1_ref[...]
        q = _silu(q)
        p = jnp.sum(q * wp2_ref[...], axis=1, keepdims=True)
        invd = 1.0 / (jnp.sqrt(d2) + 1.0)
        t = d * (p * invd)
        col3 = (lax.broadcasted_iota(jnp.int32, (1, XW), 1) == 3).astype(F32)
        t_ref[...] = t + col3

    grid = (e // BE,)
    full = lambda s: pl.BlockSpec(s, lambda i: (0,) * len(s))
    return pl.pallas_call(
        body,
        grid=grid,
        in_specs=[
            pl.BlockSpec((BE, H), lambda i: (i, 0)),
            pl.BlockSpec((BE, XW), lambda i: (i, 0)),
            pl.BlockSpec((BE, XW), lambda i: (i, 0)),
            pl.BlockSpec((BE, de), lambda i: (i, 0)),
            full((8, H)),
            full((H, 64)),
            full((1, 64)),
            full((1, 64)),
        ],
        out_specs=[
            pl.BlockSpec((BE, H), lambda i: (i, 0)),
            pl.BlockSpec((BE, XW), lambda i: (i, 0)),
        ],
        out_shape=[
            jax.ShapeDtypeStruct((e, H), F32),
            jax.ShapeDtypeStruct((e, XW), F32),
        ],
        interpret=interpret,
    )(gsum, xs, xd, ea, wx, wp1, bp1, wp2)


def _tc_node(h, xpad, magg2, agg2, wf1a, wf1b, bf1, wf2, bf2,
             w1d=None, w1s=None, last=False, interpret=False):
    n = h.shape[0]

    def body(*refs):
        if last:
            (h_ref, xp_ref, mg_ref, ag_ref, wf1a_ref, wf1b_ref, bf1_ref,
             wf2_ref, bf2_ref, ho_ref, xo_ref) = refs
        else:
            (h_ref, xp_ref, mg_ref, ag_ref, wf1a_ref, wf1b_ref, bf1_ref,
             wf2_ref, bf2_ref, w1d_ref, w1s_ref,
             ho_ref, xo_ref, hd_ref, hs_ref) = refs
        magg = mg_ref[0] + mg_ref[1]
        a = ag_ref[0] + ag_ref[1]
        cnt = jnp.maximum(a[:, 3:4], 1.0)
        posmask = (lax.broadcasted_iota(jnp.int32, (1, XW), 1) < 3).astype(F32)
        xp = xp_ref[...] + (a * posmask) / cnt
        h_ = h_ref[...]
        f = _dot(h_, wf1a_ref[...]) + _dot(magg, wf1b_ref[...]) + bf1_ref[...]
        f = _silu(f)
        f = _dot(f, wf2_ref[...]) + bf2_ref[...]
        hn = h_ + f
        if last:
            ho_ref[...] = hn
            xo_ref[...] = xp[:, :3]
        else:
            hn = _silu(hn)
            ho_ref[...] = hn
            xo_ref[...] = xp
            hd_ref[...] = _dot(hn, w1d_ref[...])
            hs_ref[...] = _dot(hn, w1s_ref[...])

    grid = (n // BN,)
    full = lambda s: pl.BlockSpec(s, lambda i: (0,) * len(s))
    in_specs = [
        pl.BlockSpec((BN, H), lambda i: (i, 0)),
        pl.BlockSpec((BN, XW), lambda i: (i, 0)),
        pl.BlockSpec((NC, BN, H), lambda i: (0, i, 0)),
        pl.BlockSpec((NC, BN, XW), lambda i: (0, i, 0)),
        full((H, H)),
        full((H, H)),
        full((1, H)),
        full((H, H)),
        full((1, H)),
    ]
    args = [h, xpad, magg2, agg2, wf1a, wf1b, bf1, wf2, bf2]
    if last:
        out_specs = [
            pl.BlockSpec((BN, H), lambda i: (i, 0)),
            pl.BlockSpec((BN, 3), lambda i: (i, 0)),
        ]
        out_shape = [
            jax.ShapeDtypeStruct((n, H), F32),
            jax.ShapeDtypeStruct((n, 3), F32),
        ]
    else:
        in_specs += [full((H, H)), full((H, H))]
        args += [w1d, w1s]
        out_specs = [
            pl.BlockSpec((BN, H), lambda i: (i, 0)),
            pl.BlockSpec((BN, XW), lambda i: (i, 0)),
            pl.BlockSpec((BN, H), lambda i: (i, 0)),
            pl.BlockSpec((BN, H), lambda i: (i, 0)),
        ]
        out_shape = [
            jax.ShapeDtypeStruct((n, H), F32),
            jax.ShapeDtypeStruct((n, XW), F32),
            jax.ShapeDtypeStruct((n, H), F32),
            jax.ShapeDtypeStruct((n, H), F32),
        ]
    return pl.pallas_call(
        body,
        grid=grid,
        in_specs=in_specs,
        out_specs=out_specs,
        out_shape=out_shape,
        interpret=interpret,
    )(*args)


def _layer_weights(layer):
    w1 = layer["edge"][0]["w"]
    b1 = layer["edge"][0]["b"]
    de = w1.shape[0] - 2 * H - 1
    w1d = w1[0:H]
    w1s = w1[H:2 * H]
    wx = jnp.concatenate(
        [w1[2 * H:2 * H + 1], w1[2 * H + 1:], b1[None],
         jnp.zeros((8 - 2 - de, w1.shape[1]), F32)], axis=0)
    wp1 = layer["pos"][0]["w"]
    bp1 = layer["pos"][0]["b"][None]
    wp2 = layer["pos"][1]["w"].T
    wf1 = layer["feat"][0]["w"]
    wf1a = wf1[:H]
    wf1b = wf1[H:]
    bf1 = layer["feat"][0]["b"][None]
    wf2 = layer["feat"][1]["w"]
    bf2 = layer["feat"][1]["b"][None]
    return w1d, w1s, wx, wp1, bp1, wp2, wf1a, wf1b, bf1, wf2, bf2


def kernel(node_feats, positions, edge_index, edge_attributes, params):
    n = node_feats.shape[0]
    e = edge_index.shape[1]
    src = edge_index[0]
    dst = edge_index[1]
    layers = params["layers"]
    depth = len(layers)
    lw = [_layer_weights(L) for L in layers]

    h, hd, hs, xpad = _tc0(
        node_feats.reshape(n, 1), positions, params["emb"],
        params["in_embed"]["w"], params["in_embed"]["b"][None],
        lw[0][0], lw[0][1])

    z128 = jnp.zeros((n, H), F32)
    z16 = jnp.zeros((n, XW), F32)

    for i in range(depth):
        w1d, w1s, wx, wp1, bp1, wp2, wf1a, wf1b, bf1, wf2, bf2 = lw[i]
        gsum, xs, xd = _sc_gather(hd, hs, xpad, src, dst)
        m, trans = _tc_edge(gsum, xs, xd, edge_attributes, wx, wp1, bp1, wp2)
        magg2, agg2 = _sc_scatter(m, trans, dst, z128, z16)
        if i == depth - 1:
            h, x = _tc_node(h, xpad, magg2, agg2, wf1a, wf1b, bf1, wf2, bf2,
                            last=True)
        else:
            h, xpad, hd, hs = _tc_node(
                h, xpad, magg2, agg2, wf1a, wf1b, bf1, wf2, bf2,
                w1d=lw[i + 1][0], w1s=lw[i + 1][1], last=False)
    return h, x


# speedup vs baseline: 4.0116x; 1.1846x over previous
import functools

import jax
import jax.numpy as jnp
from jax import lax
from jax.experimental import pallas as pl
from jax.experimental.pallas import tpu as pltpu
from jax.experimental.pallas import tpu_sc as plsc

H = 128
XW = 16
NC = 2
NS = 16
NW = NC * NS
CHUNK = 80
BN = 1000
BE = 2000
F32 = jnp.float32


def _sigmoid(v):
    return 1.0 / (1.0 + jnp.exp(-v))


def _silu(v):
    return v * _sigmoid(v)


def _dot(a, b):
    return jnp.dot(a, b, preferred_element_type=F32)


def _sc_mesh():
    return plsc.VectorSubcoreMesh(core_axis_name="c", subcore_axis_name="s")


def _sc_gather(hd, hs, xpad, src, dst):
    n, _ = hd.shape
    e = src.shape[0]
    ew = e // NW
    nch = ew // CHUNK
    assert nch % 2 == 1 and nch >= 3

    buf = lambda: [
        pltpu.VMEM((CHUNK,), jnp.int32),
        pltpu.VMEM((CHUNK,), jnp.int32),
        pltpu.VMEM((CHUNK, H), F32),
        pltpu.VMEM((CHUNK, H), F32),
        pltpu.VMEM((CHUNK, XW), F32),
        pltpu.VMEM((CHUNK, XW), F32),
        pltpu.SemaphoreType.DMA,
    ]

    @functools.partial(
        pl.kernel,
        out_type=(
            jax.ShapeDtypeStruct((e, H), F32),
            jax.ShapeDtypeStruct((e, XW), F32),
            jax.ShapeDtypeStruct((e, XW), F32),
        ),
        mesh=_sc_mesh(),
        scratch_types=buf() + buf(),
        compiler_params=pltpu.CompilerParams(use_tc_tiling_on_sc=False),
    )
    def k(hd_h, hs_h, xp_h, src_h, dst_h, gsum_h, xs_h, xd_h, *scr):
        seta, setb = scr[:7], scr[7:]
        wid = lax.axis_index("s") * NC + lax.axis_index("c")

        def issue(c, st):
            sv, dv, rd, rs, xs, xd, sem = st
            base = wid * ew + c * CHUNK
            pltpu.sync_copy(src_h.at[pl.ds(base, CHUNK)], sv)
            pltpu.sync_copy(dst_h.at[pl.ds(base, CHUNK)], dv)
            pltpu.async_copy(hd_h.at[dv], rd, sem)
            pltpu.async_copy(hs_h.at[sv], rs, sem)
            pltpu.async_copy(xp_h.at[sv], xs, sem)
            pltpu.async_copy(xp_h.at[dv], xd, sem)

        def process(c, st):
            sv, dv, rd, rs, xs, xd, sem = st
            base = wid * ew + c * CHUNK
            pltpu.make_async_copy(hd_h.at[dv], rd, sem).wait()
            pltpu.make_async_copy(hs_h.at[sv], rs, sem).wait()
            pltpu.make_async_copy(xp_h.at[sv], xs, sem).wait()
            pltpu.make_async_copy(xp_h.at[dv], xd, sem).wait()

            def row(r, carry):
                for j in range(H // 16):
                    sl = pl.ds(j * 16, 16)
                    rd[r, sl] = rd[r, sl] + rs[r, sl]
                return carry

            lax.fori_loop(0, CHUNK, row, 0)
            pltpu.sync_copy(rd, gsum_h.at[pl.ds(base, CHUNK)])
            pltpu.sync_copy(xs, xs_h.at[pl.ds(base, CHUNK)])
            pltpu.sync_copy(xd, xd_h.at[pl.ds(base, CHUNK)])

        issue(0, seta)

        def body(i, carry):
            issue(2 * i + 1, setb)
            process(2 * i, seta)
            issue(2 * i + 2, seta)
            process(2 * i + 1, setb)
            return carry

        lax.fori_loop(0, (nch - 1) // 2, body, 0)
        process(nch - 1, seta)

    return k(hd, hs, xpad, src, dst)


def _sc_scatter(m, trans, dst, z128, z16):
    e = dst.shape[0]
    n = z128.shape[0]
    ew = e // NW
    nch = ew // CHUNK
    rpt = n // NS

    @functools.partial(
        pl.kernel,
        out_type=(
            jax.ShapeDtypeStruct((NC, n, H), F32),
            jax.ShapeDtypeStruct((NC, n, XW), F32),
        ),
        mesh=_sc_mesh(),
        scratch_types=[
            pltpu.VMEM((CHUNK, H), F32),
            pltpu.VMEM((CHUNK, XW), F32),
            pltpu.VMEM((CHUNK,), jnp.int32),
            pltpu.SemaphoreType.DMA,
            pltpu.VMEM((CHUNK, H), F32),
            pltpu.VMEM((CHUNK, XW), F32),
            pltpu.VMEM((CHUNK,), jnp.int32),
            pltpu.SemaphoreType.DMA,
            pltpu.VMEM_SHARED((n, H), F32),
            pltpu.VMEM_SHARED((n, XW), F32),
        ],
        compiler_params=pltpu.CompilerParams(use_tc_tiling_on_sc=False),
    )
    def k(m_h, t_h, dst_h, z128_h, z16_h, mago_h, aggo_h,
          m_va, t_va, dst_va, sema, m_vb, t_vb, dst_vb, semb,
          mag_sh, agg_sh):
        cid = lax.axis_index("c")
        sid = lax.axis_index("s")
        wid = sid * NC + cid
        row0 = sid * rpt
        seta = (m_va, t_va, dst_va, sema)
        setb = (m_vb, t_vb, dst_vb, semb)
        pltpu.sync_copy(z128_h.at[pl.ds(row0, rpt)], mag_sh.at[pl.ds(row0, rpt)])
        pltpu.sync_copy(z16_h.at[pl.ds(row0, rpt)], agg_sh.at[pl.ds(row0, rpt)])
        plsc.subcore_barrier()

        def issue(c, st):
            m_v, t_v, dst_v, sem = st
            base = wid * ew + c * CHUNK
            pltpu.async_copy(dst_h.at[pl.ds(base, CHUNK)], dst_v, sem)
            pltpu.async_copy(m_h.at[pl.ds(base, CHUNK)], m_v, sem)
            pltpu.async_copy(t_h.at[pl.ds(base, CHUNK)], t_v, sem)

        def process(c, st):
            m_v, t_v, dst_v, sem = st
            base = wid * ew + c * CHUNK
            pltpu.make_async_copy(dst_h.at[pl.ds(base, CHUNK)], dst_v, sem).wait()
            pltpu.make_async_copy(m_h.at[pl.ds(base, CHUNK)], m_v, sem).wait()
            pltpu.make_async_copy(t_h.at[pl.ds(base, CHUNK)], t_v, sem).wait()
            pltpu.sync_copy(m_v, mag_sh.at[dst_v], add=True)
            pltpu.sync_copy(t_v, agg_sh.at[dst_v], add=True)

        issue(0, seta)

        def body(i, carry):
            issue(2 * i + 1, setb)
            process(2 * i, seta)
            issue(2 * i + 2, seta)
            process(2 * i + 1, setb)
            return carry

        lax.fori_loop(0, (nch - 1) // 2, body, 0)
        process(nch - 1, seta)
        plsc.subcore_barrier()
        pltpu.sync_copy(mag_sh.at[pl.ds(row0, rpt)],
                        mago_h.at[cid, pl.ds(row0, rpt)])
        pltpu.sync_copy(agg_sh.at[pl.ds(row0, rpt)],
                        aggo_h.at[cid, pl.ds(row0, rpt)])

    return k(m, trans, dst, z128, z16)


def _tc0(feats2d, x, emb, wemb, bemb, w1d, w1s, interpret=False):
    n = feats2d.shape[0]
    nv = emb.shape[0]

    def body(f_ref, x_ref, emb_ref, wemb_ref, bemb_ref, w1d_ref, w1s_ref,
             h_ref, hd_ref, hs_ref, xp_ref):
        f = f_ref[...]
        iota = lax.broadcasted_iota(jnp.int32, (1, nv), 1)
        onehot = (f == iota).astype(F32)
        embw = _dot(emb_ref[...], wemb_ref[...])
        h = _dot(onehot, embw) + bemb_ref[...]
        h_ref[...] = h
        hd_ref[...] = _dot(h, w1d_ref[...])
        hs_ref[...] = _dot(h, w1s_ref[...])
        xx = x_ref[...]
        xp_ref[...] = jnp.concatenate(
            [xx, jnp.zeros((xx.shape[0], XW - 3), F32)], axis=1)

    grid = (n // BN,)
    full = lambda s: pl.BlockSpec(s, lambda i: (0,) * len(s))
    return pl.pallas_call(
        body,
        grid=grid,
        in_specs=[
            pl.BlockSpec((BN, 1), lambda i: (i, 0)),
            pl.BlockSpec((BN, 3), lambda i: (i, 0)),
            full((nv, H)),
            full((H, H)),
            full((1, H)),
            full((H, H)),
            full((H, H)),
        ],
        out_specs=[
            pl.BlockSpec((BN, H), lambda i: (i, 0)),
            pl.BlockSpec((BN, H), lambda i: (i, 0)),
            pl.BlockSpec((BN, H), lambda i: (i, 0)),
            pl.BlockSpec((BN, XW), lambda i: (i, 0)),
        ],
        out_shape=[
            jax.ShapeDtypeStruct((n, H), F32),
            jax.ShapeDtypeStruct((n, H), F32),
            jax.ShapeDtypeStruct((n, H), F32),
            jax.ShapeDtypeStruct((n, XW), F32),
        ],
        interpret=interpret,
    )(feats2d, x, emb, wemb, bemb, w1d, w1s)


def _tc_edge(gsum, xs, xd, ea, wx, wp1, bp1, wp2, interpret=False):
    e = gsum.shape[0]
    de = ea.shape[1]

    def body(gsum_ref, xs_ref, xd_ref, ea_ref, wx_ref, wp1_ref,
             bp1_ref, wp2_ref, m_ref, t_ref):
        d = xs_ref[...] - xd_ref[...]
        d2 = jnp.sum(d * d, axis=1, keepdims=True)
        ones = jnp.ones((d.shape[0], 1), F32)
        zer = jnp.zeros((d.shape[0], 8 - 2 - de), F32)
        feat = jnp.concatenate([d2, ea_ref[...], ones, zer], axis=1)
        mpre = gsum_ref[...] + _dot(feat, wx_ref[...])
        m = _silu(mpre)
        m_ref[...] = m
        q = _dot(m, wp1_ref[...]) + bp1_ref[...]
        q = _silu(q)
        p = jnp.sum(q * wp2_ref[...], axis=1, keepdims=True)
        invd = 1.0 / (jnp.sqrt(d2) + 1.0)
        t = d * (p * invd)
        col3 = (lax.broadcasted_iota(jnp.int32, (1, XW), 1) == 3).astype(F32)
        t_ref[...] = t + col3

    grid = (e // BE,)
    full = lambda s: pl.BlockSpec(s, lambda i: (0,) * len(s))
    return pl.pallas_call(
        body,
        grid=grid,
        in_specs=[
            pl.BlockSpec((BE, H), lambda i: (i, 0)),
            pl.BlockSpec((BE, XW), lambda i: (i, 0)),
            pl.BlockSpec((BE, XW), lambda i: (i, 0)),
            pl.BlockSpec((BE, de), lambda i: (i, 0)),
            full((8, H)),
            full((H, 64)),
            full((1, 64)),
            full((1, 64)),
        ],
        out_specs=[
            pl.BlockSpec((BE, H), lambda i: (i, 0)),
            pl.BlockSpec((BE, XW), lambda i: (i, 0)),
        ],
        out_shape=[
            jax.ShapeDtypeStruct((e, H), F32),
            jax.ShapeDtypeStruct((e, XW), F32),
        ],
        interpret=interpret,
    )(gsum, xs, xd, ea, wx, wp1, bp1, wp2)


def _tc_node(h, xpad, magg2, agg2, wf1a, wf1b, bf1, wf2, bf2,
             w1d=None, w1s=None, last=False, interpret=False):
    n = h.shape[0]

    def body(*refs):
        if last:
            (h_ref, xp_ref, mg_ref, ag_ref, wf1a_ref, wf1b_ref, bf1_ref,
             wf2_ref, bf2_ref, ho_ref, xo_ref) = refs
        else:
            (h_ref, xp_ref, mg_ref, ag_ref, wf1a_ref, wf1b_ref, bf1_ref,
             wf2_ref, bf2_ref, w1d_ref, w1s_ref,
             ho_ref, xo_ref, hd_ref, hs_ref) = refs
        magg = mg_ref[0] + mg_ref[1]
        a = ag_ref[0] + ag_ref[1]
        cnt = jnp.maximum(a[:, 3:4], 1.0)
        posmask = (lax.broadcasted_iota(jnp.int32, (1, XW), 1) < 3).astype(F32)
        xp = xp_ref[...] + (a * posmask) / cnt
        h_ = h_ref[...]
        f = _dot(h_, wf1a_ref[...]) + _dot(magg, wf1b_ref[...]) + bf1_ref[...]
        f = _silu(f)
        f = _dot(f, wf2_ref[...]) + bf2_ref[...]
        hn = h_ + f
        if last:
            ho_ref[...] = hn
            xo_ref[...] = xp[:, :3]
        else:
            hn = _silu(hn)
            ho_ref[...] = hn
            xo_ref[...] = xp
            hd_ref[...] = _dot(hn, w1d_ref[...])
            hs_ref[...] = _dot(hn, w1s_ref[...])

    grid = (n // BN,)
    full = lambda s: pl.BlockSpec(s, lambda i: (0,) * len(s))
    in_specs = [
        pl.BlockSpec((BN, H), lambda i: (i, 0)),
        pl.BlockSpec((BN, XW), lambda i: (i, 0)),
        pl.BlockSpec((NC, BN, H), lambda i: (0, i, 0)),
        pl.BlockSpec((NC, BN, XW), lambda i: (0, i, 0)),
        full((H, H)),
        full((H, H)),
        full((1, H)),
        full((H, H)),
        full((1, H)),
    ]
    args = [h, xpad, magg2, agg2, wf1a, wf1b, bf1, wf2, bf2]
    if last:
        out_specs = [
            pl.BlockSpec((BN, H), lambda i: (i, 0)),
            pl.BlockSpec((BN, 3), lambda i: (i, 0)),
        ]
        out_shape = [
            jax.ShapeDtypeStruct((n, H), F32),
            jax.ShapeDtypeStruct((n, 3), F32),
        ]
    else:
        in_specs += [full((H, H)), full((H, H))]
        args += [w1d, w1s]
        out_specs = [
            pl.BlockSpec((BN, H), lambda i: (i, 0)),
            pl.BlockSpec((BN, XW), lambda i: (i, 0)),
            pl.BlockSpec((BN, H), lambda i: (i, 0)),
            pl.BlockSpec((BN, H), lambda i: (i, 0)),
        ]
        out_shape = [
            jax.ShapeDtypeStruct((n, H), F32),
            jax.ShapeDtypeStruct((n, XW), F32),
            jax.ShapeDtypeStruct((n, H), F32),
            jax.ShapeDtypeStruct((n, H), F32),
        ]
    return pl.pallas_call(
        body,
        grid=grid,
        in_specs=in_specs,
        out_specs=out_specs,
        out_shape=out_shape,
        interpret=interpret,
    )(*args)


def _layer_weights(layer):
    w1 = layer["edge"][0]["w"]
    b1 = layer["edge"][0]["b"]
    de = w1.shape[0] - 2 * H - 1
    w1d = w1[0:H]
    w1s = w1[H:2 * H]
    wx = jnp.concatenate(
        [w1[2 * H:2 * H + 1], w1[2 * H + 1:], b1[None],
         jnp.zeros((8 - 2 - de, w1.shape[1]), F32)], axis=0)
    wp1 = layer["pos"][0]["w"]
    bp1 = layer["pos"][0]["b"][None]
    wp2 = layer["pos"][1]["w"].T
    wf1 = layer["feat"][0]["w"]
    wf1a = wf1[:H]
    wf1b = wf1[H:]
    bf1 = layer["feat"][0]["b"][None]
    wf2 = layer["feat"][1]["w"]
    bf2 = layer["feat"][1]["b"][None]
    return w1d, w1s, wx, wp1, bp1, wp2, wf1a, wf1b, bf1, wf2, bf2


def kernel(node_feats, positions, edge_index, edge_attributes, params):
    n = node_feats.shape[0]
    e = edge_index.shape[1]
    src = edge_index[0]
    dst = edge_index[1]
    layers = params["layers"]
    depth = len(layers)
    lw = [_layer_weights(L) for L in layers]

    h, hd, hs, xpad = _tc0(
        node_feats.reshape(n, 1), positions, params["emb"],
        params["in_embed"]["w"], params["in_embed"]["b"][None],
        lw[0][0], lw[0][1])

    z128 = jnp.zeros((n, H), F32)
    z16 = jnp.zeros((n, XW), F32)

    for i in range(depth):
        w1d, w1s, wx, wp1, bp1, wp2, wf1a, wf1b, bf1, wf2, bf2 = lw[i]
        gsum, xs, xd = _sc_gather(hd, hs, xpad, src, dst)
        m, trans = _tc_edge(gsum, xs, xd, edge_attributes, wx, wp1, bp1, wp2)
        magg2, agg2 = _sc_scatter(m, trans, dst, z128, z16)
        if i == depth - 1:
            h, x = _tc_node(h, xpad, magg2, agg2, wf1a, wf1b, bf1, wf2, bf2,
                            last=True)
        else:
            h, xpad, hd, hs = _tc_node(
                h, xpad, magg2, agg2, wf1a, wf1b, bf1, wf2, bf2,
                w1d=lw[i + 1][0], w1s=lw[i + 1][1], last=False)
    return h, x


# speedup vs baseline: 4.4104x; 1.0994x over previous
import functools

import jax
import jax.numpy as jnp
from jax import lax
from jax.experimental import pallas as pl
from jax.experimental.pallas import tpu as pltpu
from jax.experimental.pallas import tpu_sc as plsc

H = 128
XW = 16
NC = 2
NS = 16
NW = NC * NS
CHUNK = 80
BN = 1000
F32 = jnp.float32


def _sigmoid(v):
    return 1.0 / (1.0 + jnp.exp(-v))


def _silu(v):
    return v * _sigmoid(v)


def _dot(a, b):
    return jnp.dot(a, b, preferred_element_type=F32)


def _sc_mesh():
    return plsc.VectorSubcoreMesh(core_axis_name="c", subcore_axis_name="s")


def _run_pipeline(issue, process, nch):
    assert nch >= 3

    def body(i, carry):
        issue(2 * i + 1, 1)
        process(2 * i, 0)
        issue(2 * i + 2, 0)
        process(2 * i + 1, 1)
        return carry

    issue(0, 0)
    if nch % 2 == 1:
        lax.fori_loop(0, (nch - 1) // 2, body, 0)
        process(nch - 1, 0)
    else:
        lax.fori_loop(0, (nch - 2) // 2, body, 0)
        issue(nch - 1, 1)
        process(nch - 2, 0)
        process(nch - 1, 1)


def _sc_gather(hd, hs, xpad, src, dst):
    n, _ = hd.shape
    e = src.shape[0]
    ew = e // NW
    nch = ew // CHUNK
    assert e % (NW * CHUNK) == 0 and nch >= 3

    buf = lambda: [
        pltpu.VMEM((CHUNK,), jnp.int32),
        pltpu.VMEM((CHUNK,), jnp.int32),
        pltpu.VMEM((CHUNK, H), F32),
        pltpu.VMEM((CHUNK, H), F32),
        pltpu.VMEM((CHUNK, XW), F32),
        pltpu.VMEM((CHUNK, XW), F32),
        pltpu.SemaphoreType.DMA,
    ]

    @functools.partial(
        pl.kernel,
        out_type=(
            jax.ShapeDtypeStruct((e, H), F32),
            jax.ShapeDtypeStruct((e, XW), F32),
            jax.ShapeDtypeStruct((e, XW), F32),
        ),
        mesh=_sc_mesh(),
        scratch_types=buf() + buf(),
        compiler_params=pltpu.CompilerParams(use_tc_tiling_on_sc=False),
    )
    def k(hd_h, hs_h, xp_h, src_h, dst_h, gsum_h, xs_h, xd_h, *scr):
        sets = (scr[:7], scr[7:])
        wid = lax.axis_index("s") * NC + lax.axis_index("c")

        def issue(c, w):
            sv, dv, rd, rs, xs, xd, sem = sets[w]
            base = wid * ew + c * CHUNK
            pltpu.sync_copy(src_h.at[pl.ds(base, CHUNK)], sv)
            pltpu.sync_copy(dst_h.at[pl.ds(base, CHUNK)], dv)
            pltpu.async_copy(hd_h.at[dv], rd, sem)
            pltpu.async_copy(hs_h.at[sv], rs, sem)
            pltpu.async_copy(xp_h.at[sv], xs, sem)
            pltpu.async_copy(xp_h.at[dv], xd, sem)

        def process(c, w):
            sv, dv, rd, rs, xs, xd, sem = sets[w]
            base = wid * ew + c * CHUNK
            pltpu.make_async_copy(hd_h.at[dv], rd, sem).wait()
            pltpu.make_async_copy(hs_h.at[sv], rs, sem).wait()
            pltpu.make_async_copy(xp_h.at[sv], xs, sem).wait()
            pltpu.make_async_copy(xp_h.at[dv], xd, sem).wait()

            def row(r, carry):
                for j in range(H // 16):
                    sl = pl.ds(j * 16, 16)
                    rd[r, sl] = rd[r, sl] + rs[r, sl]
                return carry

            lax.fori_loop(0, CHUNK, row, 0)
            pltpu.sync_copy(rd, gsum_h.at[pl.ds(base, CHUNK)])
            pltpu.sync_copy(xs, xs_h.at[pl.ds(base, CHUNK)])
            pltpu.sync_copy(xd, xd_h.at[pl.ds(base, CHUNK)])

        _run_pipeline(issue, process, nch)

    return k(hd, hs, xpad, src, dst)


def _sc_scatter(m, trans, dst, z128, z16):
    e = dst.shape[0]
    n = z128.shape[0]
    ew = e // NW
    nch = ew // CHUNK
    assert e % (NW * CHUNK) == 0 and nch >= 3
    rpt = n // NS

    @functools.partial(
        pl.kernel,
        out_type=(
            jax.ShapeDtypeStruct((NC, n, H), F32),
            jax.ShapeDtypeStruct((NC, n, XW), F32),
        ),
        mesh=_sc_mesh(),
        scratch_types=[
            pltpu.VMEM((CHUNK, H), F32),
            pltpu.VMEM((CHUNK, XW), F32),
            pltpu.VMEM((CHUNK,), jnp.int32),
            pltpu.SemaphoreType.DMA,
            pltpu.VMEM((CHUNK, H), F32),
            pltpu.VMEM((CHUNK, XW), F32),
            pltpu.VMEM((CHUNK,), jnp.int32),
            pltpu.SemaphoreType.DMA,
            pltpu.VMEM_SHARED((n, H), F32),
            pltpu.VMEM_SHARED((n, XW), F32),
        ],
        compiler_params=pltpu.CompilerParams(use_tc_tiling_on_sc=False),
    )
    def k(m_h, t_h, dst_h, z128_h, z16_h, mago_h, aggo_h,
          m_va, t_va, dst_va, sema, m_vb, t_vb, dst_vb, semb,
          mag_sh, agg_sh):
        cid = lax.axis_index("c")
        sid = lax.axis_index("s")
        wid = sid * NC + cid
        row0 = sid * rpt
        sets = ((m_va, t_va, dst_va, sema), (m_vb, t_vb, dst_vb, semb))
        pltpu.sync_copy(z128_h.at[pl.ds(row0, rpt)], mag_sh.at[pl.ds(row0, rpt)])
        pltpu.sync_copy(z16_h.at[pl.ds(row0, rpt)], agg_sh.at[pl.ds(row0, rpt)])
        plsc.subcore_barrier()

        def issue(c, w):
            m_v, t_v, dst_v, sem = sets[w]
            base = wid * ew + c * CHUNK
            pltpu.async_copy(dst_h.at[pl.ds(base, CHUNK)], dst_v, sem)
            pltpu.async_copy(m_h.at[pl.ds(base, CHUNK)], m_v, sem)
            pltpu.async_copy(t_h.at[pl.ds(base, CHUNK)], t_v, sem)

        def process(c, w):
            m_v, t_v, dst_v, sem = sets[w]
            base = wid * ew + c * CHUNK
            pltpu.make_async_copy(dst_h.at[pl.ds(base, CHUNK)], dst_v, sem).wait()
            pltpu.make_async_copy(m_h.at[pl.ds(base, CHUNK)], m_v, sem).wait()
            pltpu.make_async_copy(t_h.at[pl.ds(base, CHUNK)], t_v, sem).wait()
            pltpu.sync_copy(m_v, mag_sh.at[dst_v], add=True)
            pltpu.sync_copy(t_v, agg_sh.at[dst_v], add=True)

        _run_pipeline(issue, process, nch)
        plsc.subcore_barrier()
        pltpu.sync_copy(mag_sh.at[pl.ds(row0, rpt)],
                        mago_h.at[cid, pl.ds(row0, rpt)])
        pltpu.sync_copy(agg_sh.at[pl.ds(row0, rpt)],
                        aggo_h.at[cid, pl.ds(row0, rpt)])

    return k(m, trans, dst, z128, z16)


def _tc0(feats2d, x, emb, wemb, bemb, w1d, w1s, interpret=False):
    n = feats2d.shape[0]
    nv = emb.shape[0]

    def body(f_ref, x_ref, emb_ref, wemb_ref, bemb_ref, w1d_ref, w1s_ref,
             h_ref, hd_ref, hs_ref, xp_ref):
        f = f_ref[...]
        iota = lax.broadcasted_iota(jnp.int32, (1, nv), 1)
        onehot = (f == iota).astype(F32)
        embw = _dot(emb_ref[...], wemb_ref[...])
        h = _dot(onehot, embw) + bemb_ref[...]
        h_ref[...] = h
        hd_ref[...] = _dot(h, w1d_ref[...])
        hs_ref[...] = _dot(h, w1s_ref[...])
        xx = x_ref[...]
        xp_ref[...] = jnp.concatenate(
            [xx, jnp.zeros((xx.shape[0], XW - 3), F32)], axis=1)

    grid = (n // BN,)
    full = lambda s: pl.BlockSpec(s, lambda i: (0,) * len(s))
    return pl.pallas_call(
        body,
        grid=grid,
        in_specs=[
            pl.BlockSpec((BN, 1), lambda i: (i, 0)),
            pl.BlockSpec((BN, 3), lambda i: (i, 0)),
            full((nv, H)),
            full((H, H)),
            full((1, H)),
            full((H, H)),
            full((H, H)),
        ],
        out_specs=[
            pl.BlockSpec((BN, H), lambda i: (i, 0)),
            pl.BlockSpec((BN, H), lambda i: (i, 0)),
            pl.BlockSpec((BN, H), lambda i: (i, 0)),
            pl.BlockSpec((BN, XW), lambda i: (i, 0)),
        ],
        out_shape=[
            jax.ShapeDtypeStruct((n, H), F32),
            jax.ShapeDtypeStruct((n, H), F32),
            jax.ShapeDtypeStruct((n, H), F32),
            jax.ShapeDtypeStruct((n, XW), F32),
        ],
        interpret=interpret,
    )(feats2d, x, emb, wemb, bemb, w1d, w1s)


def _tc_edge(gsum, xs, xd, ea, wx, wp1, bp1, wp2, interpret=False):
    e = gsum.shape[0]
    de = ea.shape[1]
    be = e // 80
    assert e % 80 == 0 and be % 8 == 0

    def body(gsum_ref, xs_ref, xd_ref, ea_ref, wx_ref, wp1_ref,
             bp1_ref, wp2_ref, m_ref, t_ref):
        d = xs_ref[...] - xd_ref[...]
        d2 = jnp.sum(d * d, axis=1, keepdims=True)
        ones = jnp.ones((d.shape[0], 1), F32)
        zer = jnp.zeros((d.shape[0], 8 - 2 - de), F32)
        feat = jnp.concatenate([d2, ea_ref[...], ones, zer], axis=1)
        mpre = gsum_ref[...] + _dot(feat, wx_ref[...])
        m = _silu(mpre)
        m_ref[...] = m
        q = _dot(m, wp1_ref[...]) + bp1_ref[...]
        q = _silu(q)
        p = jnp.sum(q * wp2_ref[...], axis=1, keepdims=True)
        invd = 1.0 / (jnp.sqrt(d2) + 1.0)
        t = d * (p * invd)
        col3 = (lax.broadcasted_iota(jnp.int32, (1, XW), 1) == 3).astype(F32)
        t_ref[...] = t + col3

    grid = (e // be,)
    full = lambda s: pl.BlockSpec(s, lambda i: (0,) * len(s))
    return pl.pallas_call(
        body,
        grid=grid,
        in_specs=[
            pl.BlockSpec((be, H), lambda i: (i, 0)),
            pl.BlockSpec((be, XW), lambda i: (i, 0)),
            pl.BlockSpec((be, XW), lambda i: (i, 0)),
            pl.BlockSpec((be, de), lambda i: (i, 0)),
            full((8, H)),
            full((H, 64)),
            full((1, 64)),
            full((1, 64)),
        ],
        out_specs=[
            pl.BlockSpec((be, H), lambda i: (i, 0)),
            pl.BlockSpec((be, XW), lambda i: (i, 0)),
        ],
        out_shape=[
            jax.ShapeDtypeStruct((e, H), F32),
            jax.ShapeDtypeStruct((e, XW), F32),
        ],
        interpret=interpret,
    )(gsum, xs, xd, ea, wx, wp1, bp1, wp2)


def _tc_node(h, xpad, maggs, aggs, wf1a, wf1b, bf1, wf2, bf2,
             w1d=None, w1s=None, last=False, interpret=False):
    n = h.shape[0]
    nh = len(maggs)

    def body(*refs):
        mg_refs = refs[2:2 + nh]
        ag_refs = refs[2 + nh:2 + 2 * nh]
        rest = refs[:2] + refs[2 + 2 * nh:]
        if last:
            (h_ref, xp_ref, wf1a_ref, wf1b_ref, bf1_ref,
             wf2_ref, bf2_ref, ho_ref, xo_ref) = rest
        else:
            (h_ref, xp_ref, wf1a_ref, wf1b_ref, bf1_ref,
             wf2_ref, bf2_ref, w1d_ref, w1s_ref,
             ho_ref, xo_ref, hd_ref, hs_ref) = rest
        magg = sum(r[0] + r[1] for r in mg_refs)
        a = sum(r[0] + r[1] for r in ag_refs)
        cnt = jnp.maximum(a[:, 3:4], 1.0)
        posmask = (lax.broadcasted_iota(jnp.int32, (1, XW), 1) < 3).astype(F32)
        xp = xp_ref[...] + (a * posmask) / cnt
        h_ = h_ref[...]
        f = _dot(h_, wf1a_ref[...]) + _dot(magg, wf1b_ref[...]) + bf1_ref[...]
        f = _silu(f)
        f = _dot(f, wf2_ref[...]) + bf2_ref[...]
        hn = h_ + f
        if last:
            ho_ref[...] = hn
            xo_ref[...] = xp[:, :3]
        else:
            hn = _silu(hn)
            ho_ref[...] = hn
            xo_ref[...] = xp
            hd_ref[...] = _dot(hn, w1d_ref[...])
            hs_ref[...] = _dot(hn, w1s_ref[...])

    grid = (n // BN,)
    full = lambda s: pl.BlockSpec(s, lambda i: (0,) * len(s))
    in_specs = (
        [pl.BlockSpec((BN, H), lambda i: (i, 0)),
         pl.BlockSpec((BN, XW), lambda i: (i, 0))]
        + [pl.BlockSpec((NC, BN, H), lambda i: (0, i, 0))] * nh
        + [pl.BlockSpec((NC, BN, XW), lambda i: (0, i, 0))] * nh
        + [full((H, H)), full((H, H)), full((1, H)), full((H, H)),
           full((1, H))]
    )
    args = [h, xpad] + list(maggs) + list(aggs) + [wf1a, wf1b, bf1, wf2, bf2]
    if last:
        out_specs = [
            pl.BlockSpec((BN, H), lambda i: (i, 0)),
            pl.BlockSpec((BN, 3), lambda i: (i, 0)),
        ]
        out_shape = [
            jax.ShapeDtypeStruct((n, H), F32),
            jax.ShapeDtypeStruct((n, 3), F32),
        ]
    else:
        in_specs += [full((H, H)), full((H, H))]
        args += [w1d, w1s]
        out_specs = [
            pl.BlockSpec((BN, H), lambda i: (i, 0)),
            pl.BlockSpec((BN, XW), lambda i: (i, 0)),
            pl.BlockSpec((BN, H), lambda i: (i, 0)),
            pl.BlockSpec((BN, H), lambda i: (i, 0)),
        ]
        out_shape = [
            jax.ShapeDtypeStruct((n, H), F32),
            jax.ShapeDtypeStruct((n, XW), F32),
            jax.ShapeDtypeStruct((n, H), F32),
            jax.ShapeDtypeStruct((n, H), F32),
        ]
    return pl.pallas_call(
        body,
        grid=grid,
        in_specs=in_specs,
        out_specs=out_specs,
        out_shape=out_shape,
        interpret=interpret,
    )(*args)


def _layer_weights(layer):
    w1 = layer["edge"][0]["w"]
    b1 = layer["edge"][0]["b"]
    de = w1.shape[0] - 2 * H - 1
    w1d = w1[0:H]
    w1s = w1[H:2 * H]
    wx = jnp.concatenate(
        [w1[2 * H:2 * H + 1], w1[2 * H + 1:], b1[None],
         jnp.zeros((8 - 2 - de, w1.shape[1]), F32)], axis=0)
    wp1 = layer["pos"][0]["w"]
    bp1 = layer["pos"][0]["b"][None]
    wp2 = layer["pos"][1]["w"].T
    wf1 = layer["feat"][0]["w"]
    wf1a = wf1[:H]
    wf1b = wf1[H:]
    bf1 = layer["feat"][0]["b"][None]
    wf2 = layer["feat"][1]["w"]
    bf2 = layer["feat"][1]["b"][None]
    return w1d, w1s, wx, wp1, bp1, wp2, wf1a, wf1b, bf1, wf2, bf2


def kernel(node_feats, positions, edge_index, edge_attributes, params):
    n = node_feats.shape[0]
    e = edge_index.shape[1]
    src = edge_index[0]
    dst = edge_index[1]
    layers = params["layers"]
    depth = len(layers)
    lw = [_layer_weights(L) for L in layers]

    h, hd, hs, xpad = _tc0(
        node_feats.reshape(n, 1), positions, params["emb"],
        params["in_embed"]["w"], params["in_embed"]["b"][None],
        lw[0][0], lw[0][1])

    z128 = jnp.zeros((n, H), F32)
    z16 = jnp.zeros((n, XW), F32)

    unit = NW * CHUNK
    assert e % unit == 0
    ea_half = (e // unit) // 2 * unit
    bounds = [(0, ea_half), (ea_half, e)]
    srcs = [src[a:b] for a, b in bounds]
    dsts = [dst[a:b] for a, b in bounds]
    eats = [edge_attributes[a:b] for a, b in bounds]

    for i in range(depth):
        w1d, w1s, wx, wp1, bp1, wp2, wf1a, wf1b, bf1, wf2, bf2 = lw[i]
        gat = [_sc_gather(hd, hs, xpad, srcs[j], dsts[j]) for j in range(2)]
        edg = [_tc_edge(gat[j][0], gat[j][1], gat[j][2], eats[j],
                        wx, wp1, bp1, wp2) for j in range(2)]
        sca = [_sc_scatter(edg[j][0], edg[j][1], dsts[j], z128, z16)
               for j in range(2)]
        maggs = [sca[0][0], sca[1][0]]
        aggs = [sca[0][1], sca[1][1]]
        if i == depth - 1:
            h, x = _tc_node(h, xpad, maggs, aggs, wf1a, wf1b, bf1, wf2, bf2,
                            last=True)
        else:
            h, xpad, hd, hs = _tc_node(
                h, xpad, maggs, aggs, wf1a, wf1b, bf1, wf2, bf2,
                w1d=lw[i + 1][0], w1s=lw[i + 1][1], last=False)
    return h, x


# speedup vs baseline: 4.5004x; 1.0204x over previous
import functools

import jax
import jax.numpy as jnp
from jax import lax
from jax.experimental import pallas as pl
from jax.experimental.pallas import tpu as pltpu
from jax.experimental.pallas import tpu_sc as plsc

H = 128
XW = 16
NC = 2
NS = 16
NW = NC * NS
CHUNK = 80
BN = 1000
F32 = jnp.float32


def _sigmoid(v):
    return 1.0 / (1.0 + jnp.exp(-v))


def _silu(v):
    return v * _sigmoid(v)


def _dot(a, b):
    return jnp.dot(a, b, preferred_element_type=F32)


def _sc_mesh():
    return plsc.VectorSubcoreMesh(core_axis_name="c", subcore_axis_name="s")


def _run_pipeline(issue, process, nch):
    assert nch >= 3

    def body(i, carry):
        issue(2 * i + 1, 1)
        process(2 * i, 0)
        issue(2 * i + 2, 0)
        process(2 * i + 1, 1)
        return carry

    issue(0, 0)
    if nch % 2 == 1:
        lax.fori_loop(0, (nch - 1) // 2, body, 0)
        process(nch - 1, 0)
    else:
        lax.fori_loop(0, (nch - 2) // 2, body, 0)
        issue(nch - 1, 1)
        process(nch - 2, 0)
        process(nch - 1, 1)


def _sc_gather128(hd, hs, src, dst):
    e = src.shape[0]
    ew = e // NW
    nch = ew // CHUNK
    assert e % (NW * CHUNK) == 0 and nch >= 3

    buf = lambda: [
        pltpu.VMEM((CHUNK,), jnp.int32),
        pltpu.VMEM((CHUNK,), jnp.int32),
        pltpu.VMEM((CHUNK, H), F32),
        pltpu.VMEM((CHUNK, H), F32),
        pltpu.SemaphoreType.DMA,
    ]

    @functools.partial(
        pl.kernel,
        out_type=jax.ShapeDtypeStruct((e, H), F32),
        mesh=_sc_mesh(),
        scratch_types=buf() + buf(),
        compiler_params=pltpu.CompilerParams(use_tc_tiling_on_sc=True),
    )
    def k(hd_h, hs_h, src_h, dst_h, gsum_h, *scr):
        sets = (scr[:5], scr[5:])
        wid = lax.axis_index("s") * NC + lax.axis_index("c")

        def issue(c, w):
            sv, dv, rd, rs, sem = sets[w]
            base = wid * ew + c * CHUNK
            pltpu.sync_copy(src_h.at[pl.ds(base, CHUNK)], sv)
            pltpu.sync_copy(dst_h.at[pl.ds(base, CHUNK)], dv)
            pltpu.async_copy(hd_h.at[dv], rd, sem)
            pltpu.async_copy(hs_h.at[sv], rs, sem)

        def process(c, w):
            sv, dv, rd, rs, sem = sets[w]
            base = wid * ew + c * CHUNK
            pltpu.make_async_copy(hd_h.at[dv], rd, sem).wait()
            pltpu.make_async_copy(hs_h.at[sv], rs, sem).wait()

            def row(r, carry):
                for j in range(H // 16):
                    sl = pl.ds(j * 16, 16)
                    rd[r, sl] = rd[r, sl] + rs[r, sl]
                return carry

            lax.fori_loop(0, CHUNK, row, 0)
            pltpu.sync_copy(rd, gsum_h.at[pl.ds(base, CHUNK)])

        _run_pipeline(issue, process, nch)

    return k(hd, hs, src, dst)


def _sc_gather16(xpad, src, dst):
    e = src.shape[0]
    ew = e // NW
    nch = ew // CHUNK
    assert e % (NW * CHUNK) == 0 and nch >= 3

    buf = lambda: [
        pltpu.VMEM((CHUNK,), jnp.int32),
        pltpu.VMEM((CHUNK,), jnp.int32),
        pltpu.VMEM((CHUNK, XW), F32),
        pltpu.VMEM((CHUNK, XW), F32),
        pltpu.SemaphoreType.DMA,
    ]

    @functools.partial(
        pl.kernel,
        out_type=jax.ShapeDtypeStruct((e, XW), F32),
        mesh=_sc_mesh(),
        scratch_types=buf() + buf(),
        compiler_params=pltpu.CompilerParams(use_tc_tiling_on_sc=False),
    )
    def k(xp_h, src_h, dst_h, d_h, *scr):
        sets = (scr[:5], scr[5:])
        wid = lax.axis_index("s") * NC + lax.axis_index("c")

        def issue(c, w):
            sv, dv, xs, xd, sem = sets[w]
            base = wid * ew + c * CHUNK
            pltpu.sync_copy(src_h.at[pl.ds(base, CHUNK)], sv)
            pltpu.sync_copy(dst_h.at[pl.ds(base, CHUNK)], dv)
            pltpu.async_copy(xp_h.at[sv], xs, sem)
            pltpu.async_copy(xp_h.at[dv], xd, sem)

        def process(c, w):
            sv, dv, xs, xd, sem = sets[w]
            base = wid * ew + c * CHUNK
            pltpu.make_async_copy(xp_h.at[sv], xs, sem).wait()
            pltpu.make_async_copy(xp_h.at[dv], xd, sem).wait()

            def row(r, carry):
                xs[r, :] = xs[r, :] - xd[r, :]
                return carry

            lax.fori_loop(0, CHUNK, row, 0)
            pltpu.sync_copy(xs, d_h.at[pl.ds(base, CHUNK)])

        _run_pipeline(issue, process, nch)

    return k(xpad, src, dst)


def _sc_scatter128(m, dst, z):
    e = dst.shape[0]
    npad = z.shape[0]
    ew = e // NW
    nch = ew // CHUNK
    assert e % (NW * CHUNK) == 0 and nch >= 3
    rpt = npad // NS
    assert npad % NS == 0 and rpt % 8 == 0

    buf = lambda: [
        pltpu.VMEM((CHUNK, H), F32),
        pltpu.VMEM((CHUNK,), jnp.int32),
        pltpu.SemaphoreType.DMA,
    ]

    @functools.partial(
        pl.kernel,
        out_type=jax.ShapeDtypeStruct((NC, npad, H), F32),
        mesh=_sc_mesh(),
        scratch_types=buf() + buf() + [pltpu.VMEM_SHARED((npad, H), F32)],
        compiler_params=pltpu.CompilerParams(use_tc_tiling_on_sc=True),
    )
    def k(m_h, dst_h, z_h, mago_h, *scr):
        sets = (scr[:3], scr[3:6])
        sh = scr[6]
        cid = lax.axis_index("c")
        sid = lax.axis_index("s")
        wid = sid * NC + cid
        row0 = sid * rpt
        pltpu.sync_copy(z_h.at[pl.ds(row0, rpt)], sh.at[pl.ds(row0, rpt)])
        plsc.subcore_barrier()

        def issue(c, w):
            m_v, dst_v, sem = sets[w]
            base = wid * ew + c * CHUNK
            pltpu.async_copy(dst_h.at[pl.ds(base, CHUNK)], dst_v, sem)
            pltpu.async_copy(m_h.at[pl.ds(base, CHUNK)], m_v, sem)

        def process(c, w):
            m_v, dst_v, sem = sets[w]
            base = wid * ew + c * CHUNK
            pltpu.make_async_copy(dst_h.at[pl.ds(base, CHUNK)], dst_v, sem).wait()
            pltpu.make_async_copy(m_h.at[pl.ds(base, CHUNK)], m_v, sem).wait()
            pltpu.sync_copy(m_v, sh.at[dst_v], add=True)

        _run_pipeline(issue, process, nch)
        plsc.subcore_barrier()
        pltpu.sync_copy(sh.at[pl.ds(row0, rpt)], mago_h.at[cid, pl.ds(row0, rpt)])

    return k(m, dst, z)


def _sc_scatter16(t, dst, z):
    e = dst.shape[0]
    n = z.shape[0]
    ew = e // NW
    nch = ew // CHUNK
    assert e % (NW * CHUNK) == 0 and nch >= 3
    rpt = n // NS
    assert n % NS == 0

    buf = lambda: [
        pltpu.VMEM((CHUNK, XW), F32),
        pltpu.VMEM((CHUNK,), jnp.int32),
        pltpu.SemaphoreType.DMA,
    ]

    @functools.partial(
        pl.kernel,
        out_type=jax.ShapeDtypeStruct((NC, n, XW), F32),
        mesh=_sc_mesh(),
        scratch_types=buf() + buf() + [pltpu.VMEM_SHARED((n, XW), F32)],
        compiler_params=pltpu.CompilerParams(use_tc_tiling_on_sc=False),
    )
    def k(t_h, dst_h, z_h, aggo_h, *scr):
        sets = (scr[:3], scr[3:6])
        sh = scr[6]
        cid = lax.axis_index("c")
        sid = lax.axis_index("s")
        wid = sid * NC + cid
        row0 = sid * rpt
        pltpu.sync_copy(z_h.at[pl.ds(row0, rpt)], sh.at[pl.ds(row0, rpt)])
        plsc.subcore_barrier()

        def issue(c, w):
            t_v, dst_v, sem = sets[w]
            base = wid * ew + c * CHUNK
            pltpu.async_copy(dst_h.at[pl.ds(base, CHUNK)], dst_v, sem)
            pltpu.async_copy(t_h.at[pl.ds(base, CHUNK)], t_v, sem)

        def process(c, w):
            t_v, dst_v, sem = sets[w]
            base = wid * ew + c * CHUNK
            pltpu.make_async_copy(dst_h.at[pl.ds(base, CHUNK)], dst_v, sem).wait()
            pltpu.make_async_copy(t_h.at[pl.ds(base, CHUNK)], t_v, sem).wait()
            pltpu.sync_copy(t_v, sh.at[dst_v], add=True)

        _run_pipeline(issue, process, nch)
        plsc.subcore_barrier()
        pltpu.sync_copy(sh.at[pl.ds(row0, rpt)], aggo_h.at[cid, pl.ds(row0, rpt)])

    return k(t, dst, z)


def _tc0(feats2d, x, emb, wemb, bemb, w1d, w1s, interpret=False):
    n = feats2d.shape[0]
    nv = emb.shape[0]

    def body(f_ref, x_ref, emb_ref, wemb_ref, bemb_ref, w1d_ref, w1s_ref,
             h_ref, hd_ref, hs_ref, xp_ref):
        f = f_ref[...]
        iota = lax.broadcasted_iota(jnp.int32, (1, nv), 1)
        onehot = (f == iota).astype(F32)
        embw = _dot(emb_ref[...], wemb_ref[...])
        h = _dot(onehot, embw) + bemb_ref[...]
        h_ref[...] = h
        hd_ref[...] = _dot(h, w1d_ref[...])
        hs_ref[...] = _dot(h, w1s_ref[...])
        xx = x_ref[...]
        xp_ref[...] = jnp.concatenate(
            [xx, jnp.zeros((xx.shape[0], XW - 3), F32)], axis=1)

    grid = (n // BN,)
    full = lambda s: pl.BlockSpec(s, lambda i: (0,) * len(s))
    return pl.pallas_call(
        body,
        grid=grid,
        in_specs=[
            pl.BlockSpec((BN, 1), lambda i: (i, 0)),
            pl.BlockSpec((BN, 3), lambda i: (i, 0)),
            full((nv, H)),
            full((H, H)),
            full((1, H)),
            full((H, H)),
            full((H, H)),
        ],
        out_specs=[
            pl.BlockSpec((BN, H), lambda i: (i, 0)),
            pl.BlockSpec((BN, H), lambda i: (i, 0)),
            pl.BlockSpec((BN, H), lambda i: (i, 0)),
            pl.BlockSpec((BN, XW), lambda i: (i, 0)),
        ],
        out_shape=[
            jax.ShapeDtypeStruct((n, H), F32),
            jax.ShapeDtypeStruct((n, H), F32),
            jax.ShapeDtypeStruct((n, H), F32),
            jax.ShapeDtypeStruct((n, XW), F32),
        ],
        interpret=interpret,
    )(feats2d, x, emb, wemb, bemb, w1d, w1s)


def _tc_edge(gsum, dpk, ea, wx, wp1, bp1, wp2, interpret=False):
    e = gsum.shape[0]
    de = ea.shape[1]
    assert e % 8 == 0
    be = next(8 * d for d in range(2048 // 8, 0, -1) if (e // 8) % d == 0)

    def body(gsum_ref, d_ref, ea_ref, wx_ref, wp1_ref,
             bp1_ref, wp2_ref, m_ref, t_ref):
        d = d_ref[...]
        d2 = jnp.sum(d * d, axis=1, keepdims=True)
        ones = jnp.ones((be, 1), F32)
        zer = jnp.zeros((be, 8 - 2 - de), F32)
        feat = jnp.concatenate([d2, ea_ref[...], ones, zer], axis=1)
        mpre = gsum_ref[...] + _dot(feat, wx_ref[...])
        m = _silu(mpre)
        m_ref[...] = m
        q = _dot(m, wp1_ref[...]) + bp1_ref[...]
        q = _silu(q)
        p = jnp.sum(q * wp2_ref[...], axis=1, keepdims=True)
        invd = 1.0 / (jnp.sqrt(d2) + 1.0)
        t = d * (p * invd)
        col3 = (lax.broadcasted_iota(jnp.int32, (1, XW), 1) == 3).astype(F32)
        t_ref[...] = t + col3

    grid = (e // be,)
    full = lambda s: pl.BlockSpec(s, lambda i: (0,) * len(s))
    return pl.pallas_call(
        body,
        grid=grid,
        in_specs=[
            pl.BlockSpec((be, H), lambda i: (i, 0)),
            pl.BlockSpec((be, XW), lambda i: (i, 0)),
            pl.BlockSpec((be, de), lambda i: (i, 0)),
            full((8, H)),
            full((H, 64)),
            full((1, 64)),
            full((1, 64)),
        ],
        out_specs=[
            pl.BlockSpec((be, H), lambda i: (i, 0)),
            pl.BlockSpec((be, XW), lambda i: (i, 0)),
        ],
        out_shape=[
            jax.ShapeDtypeStruct((e, H), F32),
            jax.ShapeDtypeStruct((e, XW), F32),
        ],
        interpret=interpret,
    )(gsum, dpk, ea, wx, wp1, bp1, wp2)


def _tc_node(h, xpad, maggs, aggs, wf1a, wf1b, bf1, wf2, bf2,
             w1d=None, w1s=None, last=False, interpret=False):
    n = h.shape[0]
    nh = len(maggs)

    def body(*refs):
        mg_refs = refs[2:2 + nh]
        ag_refs = refs[2 + nh:2 + 2 * nh]
        rest = refs[:2] + refs[2 + 2 * nh:]
        if last:
            (h_ref, xp_ref, wf1a_ref, wf1b_ref, bf1_ref,
             wf2_ref, bf2_ref, ho_ref, xo_ref) = rest
        else:
            (h_ref, xp_ref, wf1a_ref, wf1b_ref, bf1_ref,
             wf2_ref, bf2_ref, w1d_ref, w1s_ref,
             ho_ref, xo_ref, hd_ref, hs_ref) = rest
        magg = sum(r[0] + r[1] for r in mg_refs)
        a = sum(r[0] + r[1] for r in ag_refs)
        cnt = jnp.maximum(a[:, 3:4], 1.0)
        posmask = (lax.broadcasted_iota(jnp.int32, (1, XW), 1) < 3).astype(F32)
        xp = xp_ref[...] + (a * posmask) / cnt
        h_ = h_ref[...]
        f = _dot(h_, wf1a_ref[...]) + _dot(magg, wf1b_ref[...]) + bf1_ref[...]
        f = _silu(f)
        f = _dot(f, wf2_ref[...]) + bf2_ref[...]
        hn = h_ + f
        if last:
            ho_ref[...] = hn
            xo_ref[...] = xp[:, :3]
        else:
            hn = _silu(hn)
            ho_ref[...] = hn
            xo_ref[...] = xp
            hd_ref[...] = _dot(hn, w1d_ref[...])
            hs_ref[...] = _dot(hn, w1s_ref[...])

    grid = (n // BN,)
    full = lambda s: pl.BlockSpec(s, lambda i: (0,) * len(s))
    in_specs = (
        [pl.BlockSpec((BN, H), lambda i: (i, 0)),
         pl.BlockSpec((BN, XW), lambda i: (i, 0))]
        + [pl.BlockSpec((NC, BN, H), lambda i: (0, i, 0))] * nh
        + [pl.BlockSpec((NC, BN, XW), lambda i: (0, i, 0))] * nh
        + [full((H, H)), full((H, H)), full((1, H)), full((H, H)),
           full((1, H))]
    )
    args = [h, xpad] + list(maggs) + list(aggs) + [wf1a, wf1b, bf1, wf2, bf2]
    if last:
        out_specs = [
            pl.BlockSpec((BN, H), lambda i: (i, 0)),
            pl.BlockSpec((BN, 3), lambda i: (i, 0)),
        ]
        out_shape = [
            jax.ShapeDtypeStruct((n, H), F32),
            jax.ShapeDtypeStruct((n, 3), F32),
        ]
    else:
        in_specs += [full((H, H)), full((H, H))]
        args += [w1d, w1s]
        out_specs = [
            pl.BlockSpec((BN, H), lambda i: (i, 0)),
            pl.BlockSpec((BN, XW), lambda i: (i, 0)),
            pl.BlockSpec((BN, H), lambda i: (i, 0)),
            pl.BlockSpec((BN, H), lambda i: (i, 0)),
        ]
        out_shape = [
            jax.ShapeDtypeStruct((n, H), F32),
            jax.ShapeDtypeStruct((n, XW), F32),
            jax.ShapeDtypeStruct((n, H), F32),
            jax.ShapeDtypeStruct((n, H), F32),
        ]
    return pl.pallas_call(
        body,
        grid=grid,
        in_specs=in_specs,
        out_specs=out_specs,
        out_shape=out_shape,
        interpret=interpret,
    )(*args)


def _layer_weights(layer):
    w1 = layer["edge"][0]["w"]
    b1 = layer["edge"][0]["b"]
    de = w1.shape[0] - 2 * H - 1
    w1d = w1[0:H]
    w1s = w1[H:2 * H]
    wx = jnp.concatenate(
        [w1[2 * H:2 * H + 1], w1[2 * H + 1:], b1[None],
         jnp.zeros((8 - 2 - de, w1.shape[1]), F32)], axis=0)
    wp1 = layer["pos"][0]["w"]
    bp1 = layer["pos"][0]["b"][None]
    wp2 = layer["pos"][1]["w"].T
    wf1 = layer["feat"][0]["w"]
    wf1a = wf1[:H]
    wf1b = wf1[H:]
    bf1 = layer["feat"][0]["b"][None]
    wf2 = layer["feat"][1]["w"]
    bf2 = layer["feat"][1]["b"][None]
    return w1d, w1s, wx, wp1, bp1, wp2, wf1a, wf1b, bf1, wf2, bf2


def kernel(node_feats, positions, edge_index, edge_attributes, params):
    n = node_feats.shape[0]
    e = edge_index.shape[1]
    src = edge_index[0]
    dst = edge_index[1]
    layers = params["layers"]
    depth = len(layers)
    lw = [_layer_weights(L) for L in layers]

    h, hd, hs, xpad = _tc0(
        node_feats.reshape(n, 1), positions, params["emb"],
        params["in_embed"]["w"], params["in_embed"]["b"][None],
        lw[0][0], lw[0][1])

    rpt = ((n + NS - 1) // NS + 7) // 8 * 8
    npad = rpt * NS
    z128 = jnp.zeros((npad, H), F32)
    z16 = jnp.zeros((n, XW), F32)

    unit = NW * CHUNK
    assert e % unit == 0
    ea_half = (e // unit) // 2 * unit
    bounds = [(0, ea_half), (ea_half, e)]
    srcs = [src[a:b] for a, b in bounds]
    dsts = [dst[a:b] for a, b in bounds]
    eats = [edge_attributes[a:b] for a, b in bounds]

    for i in range(depth):
        w1d, w1s, wx, wp1, bp1, wp2, wf1a, wf1b, bf1, wf2, bf2 = lw[i]
        gs = [_sc_gather128(hd, hs, srcs[j], dsts[j]) for j in range(2)]
        dl = [_sc_gather16(xpad, srcs[j], dsts[j]) for j in range(2)]
        edg = [_tc_edge(gs[j], dl[j], eats[j],
                        wx, wp1, bp1, wp2) for j in range(2)]
        maggs = [_sc_scatter128(edg[j][0], dsts[j], z128) for j in range(2)]
        aggs = [_sc_scatter16(edg[j][1], dsts[j], z16) for j in range(2)]
        if i == depth - 1:
            h, x = _tc_node(h, xpad, maggs, aggs,
                            wf1a, wf1b, bf1, wf2, bf2, last=True)
        else:
            h, xpad, hd, hs = _tc_node(
                h, xpad, maggs, aggs, wf1a, wf1b, bf1, wf2, bf2,
                w1d=lw[i + 1][0], w1s=lw[i + 1][1], last=False)
    return h, x
